# baseline (speedup 1.0000x reference)
import jax
import jax.numpy as jnp
from jax import lax
from jax.experimental import pallas as pl
from jax.experimental.pallas import tpu as pltpu

N = 8192
DIM = 4096
K = 3

BLK = 256
GRID_F = N // BLK
SUB = BLK // 128


def _fused_body(h_blk, tokens_ref, hid_any, anchors_ref, meta_ref,
                norms_ref, sem):
    i = pl.program_id(0)
    x = h_blk[...]
    s = jnp.sum(x * x, axis=1)
    norms_ref[pl.ds(i * SUB, SUB), :] = s.reshape(SUB, 128)

    @pl.when(i == GRID_F - 1)
    def _():
        v = norms_ref[...]
        row = lax.broadcasted_iota(jnp.int32, v.shape, 0)
        lane = lax.broadcasted_iota(jnp.int32, v.shape, 1)
        gidx = row * 128 + lane
        big = jnp.int32(2**31 - 1)

        idxs = []
        for _ in range(K):
            m = jnp.max(v)
            cand = jnp.where(v == m, gidx, big)
            ik = jnp.min(cand)
            idxs.append(ik)
            v = jnp.where(gidx == ik, jnp.float32(-1.0), v)

        t = tokens_ref[...]
        toks = [jnp.sum(jnp.where(gidx == ik, t, 0)) for ik in idxs]

        lane8 = lax.broadcasted_iota(jnp.int32, (8, 128), 1)
        meta_ref[...] = jnp.where(lane8 == 0, toks[0],
                                  jnp.where(lane8 == 1, toks[1],
                                            jnp.where(lane8 == 2, toks[2], 0)))

        for k, ik in enumerate(idxs):
            cp = pltpu.make_async_copy(hid_any.at[pl.ds(ik, 1), :],
                                       anchors_ref.at[pl.ds(k, 1), :], sem)
            cp.start()
            cp.wait()


@jax.jit
def _run(hidden, tokens_2d):
    anchors, meta = pl.pallas_call(
        _fused_body,
        grid=(GRID_F,),
        in_specs=[
            pl.BlockSpec((BLK, DIM), lambda i: (i, 0)),
            pl.BlockSpec(memory_space=pltpu.VMEM),
            pl.BlockSpec(memory_space=pl.ANY),
        ],
        out_specs=[
            pl.BlockSpec((K, DIM), lambda i: (0, 0)),
            pl.BlockSpec((8, 128), lambda i: (0, 0)),
        ],
        out_shape=[
            jax.ShapeDtypeStruct((K, DIM), jnp.float32),
            jax.ShapeDtypeStruct((8, 128), jnp.int32),
        ],
        scratch_shapes=[
            pltpu.VMEM((N // 128, 128), jnp.float32),
            pltpu.SemaphoreType.DMA,
        ],
        compiler_params=pltpu.CompilerParams(
            dimension_semantics=("arbitrary",)),
    )(hidden, tokens_2d, hidden)
    return anchors, meta


def kernel(hidden, tokens):
    tokens_2d = tokens.astype(jnp.int32).reshape(N // 128, 128)
    anchors, meta = _run(hidden, tokens_2d)
    sel_tokens = meta[0, :K].astype(tokens.dtype)
    return anchors, sel_tokens


# speedup vs baseline: 1.0361x; 1.0361x over previous
import jax
import jax.numpy as jnp
from jax import lax
from jax.experimental import pallas as pl
from jax.experimental.pallas import tpu as pltpu

N = 8192
DIM = 4096
K = 3

BLK = 1024
GRID_F = N // BLK
SUB = BLK // 128


def _fused_body(h_blk, tokens_ref, hid_any, anchors_ref, meta_ref,
                norms_ref, sem):
    i = pl.program_id(0)
    x = h_blk[...]
    s = jnp.sum(x * x, axis=1)
    norms_ref[pl.ds(i * SUB, SUB), :] = s.reshape(SUB, 128)

    @pl.when(i == GRID_F - 1)
    def _():
        v = norms_ref[...]
        row = lax.broadcasted_iota(jnp.int32, v.shape, 0)
        lane = lax.broadcasted_iota(jnp.int32, v.shape, 1)
        gidx = row * 128 + lane
        big = jnp.int32(2**31 - 1)

        idxs = []
        for _ in range(K):
            m = jnp.max(v)
            cand = jnp.where(v == m, gidx, big)
            ik = jnp.min(cand)
            idxs.append(ik)
            v = jnp.where(gidx == ik, jnp.float32(-1.0), v)

        t = tokens_ref[...]
        toks = [jnp.sum(jnp.where(gidx == ik, t, 0)) for ik in idxs]

        lane8 = lax.broadcasted_iota(jnp.int32, (8, 128), 1)
        meta_ref[...] = jnp.where(lane8 == 0, toks[0],
                                  jnp.where(lane8 == 1, toks[1],
                                            jnp.where(lane8 == 2, toks[2], 0)))

        for k, ik in enumerate(idxs):
            cp = pltpu.make_async_copy(hid_any.at[pl.ds(ik, 1), :],
                                       anchors_ref.at[pl.ds(k, 1), :], sem)
            cp.start()
            cp.wait()


@jax.jit
def _run(hidden, tokens_2d):
    anchors, meta = pl.pallas_call(
        _fused_body,
        grid=(GRID_F,),
        in_specs=[
            pl.BlockSpec((BLK, DIM), lambda i: (i, 0)),
            pl.BlockSpec(memory_space=pltpu.VMEM),
            pl.BlockSpec(memory_space=pl.ANY),
        ],
        out_specs=[
            pl.BlockSpec((K, DIM), lambda i: (0, 0)),
            pl.BlockSpec((8, 128), lambda i: (0, 0)),
        ],
        out_shape=[
            jax.ShapeDtypeStruct((K, DIM), jnp.float32),
            jax.ShapeDtypeStruct((8, 128), jnp.int32),
        ],
        scratch_shapes=[
            pltpu.VMEM((N // 128, 128), jnp.float32),
            pltpu.SemaphoreType.DMA,
        ],
        compiler_params=pltpu.CompilerParams(
            dimension_semantics=("arbitrary",)),
    )(hidden, tokens_2d, hidden)
    return anchors, meta


def kernel(hidden, tokens):
    tokens_2d = tokens.astype(jnp.int32).reshape(N // 128, 128)
    anchors, meta = _run(hidden, tokens_2d)
    sel_tokens = meta[0, :K].astype(tokens.dtype)
    return anchors, sel_tokens


# speedup vs baseline: 1.1033x; 1.0648x over previous
import jax
import jax.numpy as jnp
from jax import lax
from jax.experimental import pallas as pl
from jax.experimental.pallas import tpu as pltpu

N = 8192
DIM = 4096
K = 3

BLK = 512
GRID_F = N // BLK
SUB = BLK // 128


def _fused_body(h_blk, tokens_ref, hid_any, anchors_ref, meta_ref,
                norms_ref, sem):
    i = pl.program_id(0)
    x = h_blk[...]
    s = jnp.sum(x * x, axis=1)
    norms_ref[pl.ds(i * SUB, SUB), :] = s.reshape(SUB, 128)

    @pl.when(i == GRID_F - 1)
    def _():
        v = norms_ref[...]
        row = lax.broadcasted_iota(jnp.int32, v.shape, 0)
        lane = lax.broadcasted_iota(jnp.int32, v.shape, 1)
        gidx = row * 128 + lane
        big = jnp.int32(2**31 - 1)

        idxs = []
        for _ in range(K):
            m = jnp.max(v)
            cand = jnp.where(v == m, gidx, big)
            ik = jnp.min(cand)
            idxs.append(ik)
            v = jnp.where(gidx == ik, jnp.float32(-1.0), v)

        t = tokens_ref[...]
        toks = [jnp.sum(jnp.where(gidx == ik, t, 0)) for ik in idxs]

        lane8 = lax.broadcasted_iota(jnp.int32, (8, 128), 1)
        meta_ref[...] = jnp.where(lane8 == 0, toks[0],
                                  jnp.where(lane8 == 1, toks[1],
                                            jnp.where(lane8 == 2, toks[2], 0)))

        cps = [pltpu.make_async_copy(hid_any.at[pl.ds(ik, 1), :],
                                     anchors_ref.at[pl.ds(k, 1), :], sem)
               for k, ik in enumerate(idxs)]
        for cp in cps:
            cp.start()
        for cp in cps:
            cp.wait()


@jax.jit
def _run(hidden, tokens_2d):
    anchors, meta = pl.pallas_call(
        _fused_body,
        grid=(GRID_F,),
        in_specs=[
            pl.BlockSpec((BLK, DIM), lambda i: (i, 0)),
            pl.BlockSpec(memory_space=pltpu.VMEM),
            pl.BlockSpec(memory_space=pl.ANY),
        ],
        out_specs=[
            pl.BlockSpec((K, DIM), lambda i: (0, 0)),
            pl.BlockSpec((8, 128), lambda i: (0, 0)),
        ],
        out_shape=[
            jax.ShapeDtypeStruct((K, DIM), jnp.float32),
            jax.ShapeDtypeStruct((8, 128), jnp.int32),
        ],
        scratch_shapes=[
            pltpu.VMEM((N // 128, 128), jnp.float32),
            pltpu.SemaphoreType.DMA,
        ],
        compiler_params=pltpu.CompilerParams(
            dimension_semantics=("arbitrary",)),
    )(hidden, tokens_2d, hidden)
    return anchors, meta


def kernel(hidden, tokens):
    tokens_2d = tokens.astype(jnp.int32).reshape(N // 128, 128)
    anchors, meta = _run(hidden, tokens_2d)
    sel_tokens = meta[0, :K].astype(tokens.dtype)
    return anchors, sel_tokens
